# baseline (speedup 1.0000x reference)
import functools

import jax
import jax.numpy as jnp
from jax import lax
from jax.experimental import pallas as pl
from jax.experimental.pallas import tpu as pltpu
from jax.experimental.pallas import tpu_sc as plsc

_V = 100000
_D = 64
_SEQ = 8192
_CHUNK = 2048


@functools.lru_cache(maxsize=None)
def _build(nbatch: int):
    info = plsc.get_sparse_core_info()
    nc, ns = info.num_cores, info.num_subcores
    nw = nc * ns
    d_per_w = _D // nw
    n_chunks = nbatch * _SEQ // _CHUNK
    mesh = plsc.VectorSubcoreMesh(core_axis_name="c", subcore_axis_name="s")

    @functools.partial(
        pl.kernel,
        mesh=mesh,
        compiler_params=pltpu.CompilerParams(
            use_tc_tiling_on_sc=True, needs_layout_passes=False),
        out_type=jax.ShapeDtypeStruct((nbatch, _D, _SEQ), jnp.float32),
        scratch_types=[
            pltpu.VMEM((_V,), jnp.float32),
            pltpu.VMEM((2, _CHUNK), jnp.int32),
            pltpu.VMEM((2, _CHUNK), jnp.float32),
            pltpu.VMEM((16,), jnp.float32),
            pltpu.SemaphoreType.DMA((2,)),
            pltpu.SemaphoreType.DMA((2,)),
        ],
    )
    def k(idx_hbm, tblt_hbm, scale_hbm, out_hbm, tbl_v, idx_v, stage_v,
          scale_v, idx_sem, out_sem):
        wid = lax.axis_index("s") * nc + lax.axis_index("c")
        pltpu.sync_copy(scale_hbm, scale_v)
        sv = scale_v[...]

        def idx_copy(u, s):
            return pltpu.make_async_copy(
                idx_hbm.at[u // 4, pl.ds((u % 4) * _CHUNK, _CHUNK)],
                idx_v.at[s], idx_sem.at[s])

        def out_copy(u, s):
            return pltpu.make_async_copy(
                stage_v.at[s],
                out_hbm.at[u // 4, c_cur[0], pl.ds((u % 4) * _CHUNK, _CHUNK)],
                out_sem.at[s])

        c_cur = [None]

        for f in range(d_per_w):
            c = wid * d_per_w + f
            c_cur[0] = c
            pltpu.sync_copy(tblt_hbm.at[c], tbl_v)
            idx_copy(0, 0).start()
            for u in range(n_chunks):
                s = u % 2
                if u + 1 < n_chunks:
                    idx_copy(u + 1, 1 - s).start()
                idx_copy(u, s).wait()
                if u >= 2:
                    out_copy(u - 2, s).wait()

                @plsc.parallel_loop(0, _CHUNK, 16, unroll=16)
                def g(o):
                    iv = idx_v[s, pl.ds(o, 16)]
                    stage_v[s, pl.ds(o, 16)] = (
                        plsc.load_gather(tbl_v, [iv]) * sv)
                out_copy(u, s).start()
            for u in range(n_chunks - 2, n_chunks):
                out_copy(u, u % 2).wait()

    return k


def kernel(token_ids, embed_weight, scale):
    nbatch, seq = token_ids.shape
    idx = token_ids.astype(jnp.int32)
    scale_vec = jnp.broadcast_to(scale.astype(jnp.float32), (16,))
    out_t = _build(nbatch)(idx, embed_weight.T, scale_vec)
    return out_t.transpose(0, 2, 1)


# speedup vs baseline: 1.0190x; 1.0190x over previous
import functools

import jax
import jax.numpy as jnp
from jax import lax
from jax.experimental import pallas as pl
from jax.experimental.pallas import tpu as pltpu
from jax.experimental.pallas import tpu_sc as plsc

_V = 100000
_D = 64
_SEQ = 8192
_CHUNK = 2048


@functools.lru_cache(maxsize=None)
def _build(nbatch: int):
    info = plsc.get_sparse_core_info()
    nc, ns = info.num_cores, info.num_subcores
    nw = nc * ns
    d_per_w = _D // nw
    n_chunks = nbatch * _SEQ // _CHUNK
    mesh = plsc.VectorSubcoreMesh(core_axis_name="c", subcore_axis_name="s")

    @functools.partial(
        pl.kernel,
        mesh=mesh,
        compiler_params=pltpu.CompilerParams(
            use_tc_tiling_on_sc=True, needs_layout_passes=False),
        out_type=jax.ShapeDtypeStruct((nbatch, _D, _SEQ), jnp.float32),
        scratch_types=[
            pltpu.VMEM((_V,), jnp.float32),
            pltpu.VMEM((2, _CHUNK), jnp.int32),
            pltpu.VMEM((2, _CHUNK), jnp.float32),
            pltpu.VMEM((16,), jnp.float32),
            pltpu.SemaphoreType.DMA((2,)),
            pltpu.SemaphoreType.DMA((2,)),
        ],
    )
    def k(idx_hbm, tblt_hbm, scale_hbm, out_hbm, tbl_v, idx_v, stage_v,
          scale_v, idx_sem, out_sem):
        wid = lax.axis_index("s") * nc + lax.axis_index("c")
        pltpu.sync_copy(scale_hbm, scale_v)
        sv = scale_v[...]

        def idx_copy(u, s):
            return pltpu.make_async_copy(
                idx_hbm.at[u // 4, pl.ds((u % 4) * _CHUNK, _CHUNK)],
                idx_v.at[s], idx_sem.at[s])

        def out_copy(u, s):
            return pltpu.make_async_copy(
                stage_v.at[s],
                out_hbm.at[u // 4, c_cur[0], pl.ds((u % 4) * _CHUNK, _CHUNK)],
                out_sem.at[s])

        c_cur = [None]

        for f in range(d_per_w):
            c = wid * d_per_w + f
            c_cur[0] = c
            pltpu.sync_copy(tblt_hbm.at[c], tbl_v)
            idx_copy(0, 0).start()
            for u in range(n_chunks):
                s = u % 2
                if u + 1 < n_chunks:
                    idx_copy(u + 1, 1 - s).start()
                idx_copy(u, s).wait()
                if u >= 2:
                    out_copy(u - 2, s).wait()

                @plsc.parallel_loop(0, _CHUNK, 16, unroll=8)
                def g(o):
                    iv = idx_v[s, pl.ds(o, 16)]
                    stage_v[s, pl.ds(o, 16)] = (
                        plsc.load_gather(tbl_v, [iv]) * sv)
                out_copy(u, s).start()
            for u in range(n_chunks - 2, n_chunks):
                out_copy(u, u % 2).wait()

    return k


def kernel(token_ids, embed_weight, scale):
    nbatch, seq = token_ids.shape
    idx = token_ids.astype(jnp.int32)
    scale_vec = jnp.broadcast_to(scale.astype(jnp.float32), (16,))
    out_t = _build(nbatch)(idx, embed_weight.T, scale_vec)
    return out_t.transpose(0, 2, 1)


# speedup vs baseline: 1.0391x; 1.0197x over previous
import functools

import jax
import jax.numpy as jnp
from jax import lax
from jax.experimental import pallas as pl
from jax.experimental.pallas import tpu as pltpu
from jax.experimental.pallas import tpu_sc as plsc

_V = 100000
_D = 64
_SEQ = 8192
_CHUNK = 2048


@functools.lru_cache(maxsize=None)
def _build(nbatch: int):
    info = plsc.get_sparse_core_info()
    nc, ns = info.num_cores, info.num_subcores
    nw = nc * ns
    d_per_w = _D // nw
    n_chunks = nbatch * _SEQ // _CHUNK
    mesh = plsc.VectorSubcoreMesh(core_axis_name="c", subcore_axis_name="s")

    @functools.partial(
        pl.kernel,
        mesh=mesh,
        compiler_params=pltpu.CompilerParams(
            use_tc_tiling_on_sc=True, needs_layout_passes=False),
        out_type=jax.ShapeDtypeStruct((nbatch, _D, _SEQ), jnp.float32),
        scratch_types=[
            pltpu.VMEM((_V,), jnp.float32),
            pltpu.VMEM((2, _CHUNK), jnp.int32),
            pltpu.VMEM((2, _CHUNK), jnp.float32),
            pltpu.VMEM((16,), jnp.float32),
            pltpu.SemaphoreType.DMA((2,)),
            pltpu.SemaphoreType.DMA((2,)),
        ],
    )
    def k(idx_hbm, tblt_hbm, scale_hbm, out_hbm, tbl_v, idx_v, stage_v,
          scale_v, idx_sem, out_sem):
        wid = lax.axis_index("s") * nc + lax.axis_index("c")
        pltpu.sync_copy(scale_hbm, scale_v)
        sv = scale_v[...]

        def idx_copy(u, s):
            return pltpu.make_async_copy(
                idx_hbm.at[pl.ds(u * _CHUNK, _CHUNK)],
                idx_v.at[s], idx_sem.at[s])

        def out_copy(u, s):
            return pltpu.make_async_copy(
                stage_v.at[s],
                out_hbm.at[u // 4, c_cur[0], pl.ds((u % 4) * _CHUNK, _CHUNK)],
                out_sem.at[s])

        c_cur = [None]

        for f in range(d_per_w):
            c = wid * d_per_w + f
            c_cur[0] = c
            pltpu.sync_copy(tblt_hbm.at[c], tbl_v)
            idx_copy(0, 0).start()
            for u in range(n_chunks):
                s = u % 2
                if u + 1 < n_chunks:
                    idx_copy(u + 1, 1 - s).start()
                idx_copy(u, s).wait()
                if u >= 2:
                    out_copy(u - 2, s).wait()

                @plsc.parallel_loop(0, _CHUNK, 16, unroll=8)
                def g(o):
                    iv = idx_v[s, pl.ds(o, 16)]
                    stage_v[s, pl.ds(o, 16)] = (
                        plsc.load_gather(tbl_v, [iv]) * sv)
                out_copy(u, s).start()
            for u in range(n_chunks - 2, n_chunks):
                out_copy(u, u % 2).wait()

    return k


def kernel(token_ids, embed_weight, scale):
    nbatch, seq = token_ids.shape
    idx = token_ids.reshape(-1).astype(jnp.int32)
    scale_vec = jnp.broadcast_to(scale.astype(jnp.float32), (16,))
    out_t = _build(nbatch)(idx, embed_weight.T, scale_vec)
    return out_t.transpose(0, 2, 1)


# speedup vs baseline: 1.1814x; 1.1369x over previous
import functools

import jax
import jax.numpy as jnp
from jax import lax
from jax.experimental import pallas as pl
from jax.experimental.pallas import tpu as pltpu
from jax.experimental.pallas import tpu_sc as plsc

_V = 100000
_D = 64
_SEQ = 8192
_CHUNK = 2048


@functools.lru_cache(maxsize=None)
def _build(nbatch: int):
    info = plsc.get_sparse_core_info()
    nc, ns = info.num_cores, info.num_subcores
    nw = nc * ns
    d_per_w = _D // nw
    n_chunks = nbatch * _SEQ // _CHUNK
    mesh = plsc.VectorSubcoreMesh(core_axis_name="c", subcore_axis_name="s")

    @functools.partial(
        pl.kernel,
        mesh=mesh,
        compiler_params=pltpu.CompilerParams(
            use_tc_tiling_on_sc=True, needs_layout_passes=False),
        out_type=jax.ShapeDtypeStruct((nbatch, _D, _SEQ), jnp.float32),
        scratch_types=[
            pltpu.VMEM((_V,), jnp.float32),
            pltpu.VMEM((2, _CHUNK), jnp.int32),
            pltpu.VMEM((2, _CHUNK), jnp.float32),
            pltpu.VMEM((16,), jnp.float32),
            pltpu.SemaphoreType.DMA((2,)),
            pltpu.SemaphoreType.DMA((2,)),
        ],
    )
    def k(idx_hbm, tblt_hbm, scale_hbm, out_hbm, tbl_v, idx_v, stage_v,
          scale_v, idx_sem, out_sem):
        wid = lax.axis_index("s") * nc + lax.axis_index("c")
        pltpu.sync_copy(scale_hbm, scale_v)
        sv = scale_v[...]

        def idx_copy(u, s):
            return pltpu.make_async_copy(
                idx_hbm.at[pl.ds(u * _CHUNK, _CHUNK)],
                idx_v.at[s], idx_sem.at[s])

        def out_copy(u, c, s):
            return pltpu.make_async_copy(
                stage_v.at[s],
                out_hbm.at[u // 4, c, pl.ds((u % 4) * _CHUNK, _CHUNK)],
                out_sem.at[s])

        for f in range(d_per_w):
            c = wid * d_per_w + f
            pltpu.sync_copy(tblt_hbm.at[c], tbl_v)
            idx_copy(0, 0).start()

            def pair(t, carry):
                for s in range(2):
                    u = 2 * t + s

                    @pl.when(u + 1 < n_chunks)
                    def _():
                        idx_copy(u + 1, 1 - s).start()

                    idx_copy(u, s).wait()

                    @pl.when(u >= 2)
                    def _():
                        out_copy(u - 2, c, s).wait()

                    @plsc.parallel_loop(0, _CHUNK, 16, unroll=8)
                    def g(o):
                        iv = idx_v[s, pl.ds(o, 16)]
                        stage_v[s, pl.ds(o, 16)] = (
                            plsc.load_gather(tbl_v, [iv]) * sv)
                    out_copy(u, c, s).start()
                return carry

            lax.fori_loop(0, n_chunks // 2, pair, 0)
            for s in range(2):
                out_copy(n_chunks - 2 + s, c, s).wait()

    return k


def kernel(token_ids, embed_weight, scale):
    nbatch, seq = token_ids.shape
    idx = token_ids.reshape(-1).astype(jnp.int32)
    scale_vec = jnp.broadcast_to(scale.astype(jnp.float32), (16,))
    out_t = _build(nbatch)(idx, embed_weight.T, scale_vec)
    return out_t.transpose(0, 2, 1)
